# Initial kernel scaffold; baseline (speedup 1.0000x reference)
import functools

import jax
import jax.numpy as jnp
from jax import lax
from jax.experimental import pallas as pl
from jax.experimental.pallas import tpu as pltpu
from jax.experimental.pallas import tpu_sc as plsc

B, C, H, W = 2, 96, 384, 384
HW = H * W
NPIX = B * HW
NC, NS, L = 2, 16, 16
NW = NC * NS
PIX_W = NPIX // NW
P = 128
CB = C // L


def _sc_gather_combine(xt, idx, w):
  mesh = plsc.VectorSubcoreMesh(core_axis_name="c", subcore_axis_name="s")

  @functools.partial(
      pl.kernel,
      out_type=jax.ShapeDtypeStruct((NPIX, C), jnp.float32),
      mesh=mesh,
      scratch_types=[
          [pltpu.VMEM((P,), jnp.int32) for _ in range(4)],
          [pltpu.VMEM((P,), jnp.float32) for _ in range(4)],
          [pltpu.VMEM((P, C), jnp.float32) for _ in range(4)],
          pltpu.VMEM((P, C), jnp.float32),
          pltpu.SemaphoreType.DMA,
      ],
  )
  def k(xt_hbm, idx_hbm, w_hbm, out_hbm, idx_vs, w_vs, row_vs, out_v, sem):
    wid = lax.axis_index("s") * NC + lax.axis_index("c")
    base = wid * PIX_W

    @pl.loop(0, PIX_W, step=P)
    def _chunk(off):
      p0 = base + off
      for k4 in range(4):
        pltpu.sync_copy(idx_hbm.at[pl.ds(k4 * NPIX + p0, P)], idx_vs[k4])
        pltpu.sync_copy(w_hbm.at[pl.ds(k4 * NPIX + p0, P)], w_vs[k4])
      copies = [
          pltpu.async_copy(xt_hbm.at[idx_vs[k4]], row_vs[k4], sem)
          for k4 in range(4)
      ]
      for cp in copies:
        cp.wait()

      @pl.loop(0, P)
      def _pix(pi):
        pidx = jnp.full((L,), pi, jnp.int32)
        ws = [plsc.load_gather(w_vs[k4], [pidx]) for k4 in range(4)]
        for cb in range(CB):
          sl = pl.ds(cb * L, L)
          acc = ws[0] * row_vs[0][pi, sl]
          acc = acc + ws[1] * row_vs[1][pi, sl]
          acc = acc + ws[2] * row_vs[2][pi, sl]
          acc = acc + ws[3] * row_vs[3][pi, sl]
          out_v[pi, sl] = acc

      pltpu.sync_copy(out_v, out_hbm.at[pl.ds(p0, P)])

  return k(xt, idx, w)


def kernel(x, warp):
  xf = x.astype(jnp.float32)
  gy = lax.broadcasted_iota(jnp.float32, (1, H, W), 1)
  gx = lax.broadcasted_iota(jnp.float32, (1, H, W), 2)
  sx = gx + warp[:, 0]
  sy = gy + warp[:, 1]
  x0 = jnp.floor(sx)
  y0 = jnp.floor(sy)
  wx = sx - x0
  wy = sy - y0

  def inb(xi, yi):
    return ((xi >= 0) & (xi <= W - 1) & (yi >= 0) & (yi <= H - 1)).astype(
        jnp.float32)

  boff = lax.broadcasted_iota(jnp.int32, (B, H, W), 0) * HW

  def flat_idx(xi, yi):
    xi_c = jnp.clip(xi, 0, W - 1).astype(jnp.int32)
    yi_c = jnp.clip(yi, 0, H - 1).astype(jnp.int32)
    return yi_c * W + xi_c + boff

  idx = jnp.stack([
      flat_idx(x0, y0), flat_idx(x0 + 1.0, y0),
      flat_idx(x0, y0 + 1.0), flat_idx(x0 + 1.0, y0 + 1.0)
  ]).reshape(4 * NPIX)
  w = jnp.stack([
      (1.0 - wx) * (1.0 - wy) * inb(x0, y0),
      wx * (1.0 - wy) * inb(x0 + 1.0, y0),
      (1.0 - wx) * wy * inb(x0, y0 + 1.0),
      wx * wy * inb(x0 + 1.0, y0 + 1.0),
  ]).reshape(4 * NPIX)

  xt = xf.transpose(0, 2, 3, 1).reshape(NPIX, C)
  xt = jnp.concatenate([xt, jnp.zeros((8, C), jnp.float32)], axis=0)
  out_t = _sc_gather_combine(xt, idx, w)
  return out_t.reshape(B, H, W, C).transpose(0, 3, 1, 2)


# speedup vs baseline: 4.5308x; 4.5308x over previous
import dataclasses
import functools

import jax
import jax.numpy as jnp
from jax import lax
from jax.experimental import pallas as pl
from jax.experimental.pallas import tpu as pltpu
from jax.experimental.pallas import tpu_sc as plsc

B, C, H, W = 2, 96, 384, 384
HW = H * W
NPIX = B * HW
NC, NS, L = 2, 16, 16
NW = NC * NS
PIX_W = NPIX // NW
P = 128
CB = C // L


def _sc_gather_combine(xt, idx, w):
  mesh = plsc.VectorSubcoreMesh(core_axis_name="c", subcore_axis_name="s")
  cp = pltpu.CompilerParams()
  if "needs_layout_passes" in pltpu.CompilerParams.__dataclass_fields__:
    cp = dataclasses.replace(cp, needs_layout_passes=False)
  if "use_tc_tiling_on_sc" in pltpu.CompilerParams.__dataclass_fields__:
    cp = dataclasses.replace(cp, use_tc_tiling_on_sc=False)

  @functools.partial(
      pl.kernel,
      compiler_params=cp,
      out_type=jax.ShapeDtypeStruct((NPIX, C), jnp.float32),
      mesh=mesh,
      scratch_types=[
          [pltpu.VMEM((P,), jnp.int32) for _ in range(4)],
          [pltpu.VMEM((P,), jnp.float32) for _ in range(4)],
          [pltpu.VMEM((P, C), jnp.float32) for _ in range(4)],
          pltpu.VMEM((P, C), jnp.float32),
          pltpu.SemaphoreType.DMA,
      ],
  )
  def k(xt_hbm, idx_hbm, w_hbm, out_hbm, idx_vs, w_vs, row_vs, out_v, sem):
    wid = lax.axis_index("s") * NC + lax.axis_index("c")
    base = wid * PIX_W

    @pl.loop(0, PIX_W, step=P)
    def _chunk(off):
      p0 = base + off
      for k4 in range(4):
        pltpu.sync_copy(idx_hbm.at[pl.ds(k4 * NPIX + p0, P)], idx_vs[k4])
        pltpu.sync_copy(w_hbm.at[pl.ds(k4 * NPIX + p0, P)], w_vs[k4])
      copies = [
          pltpu.async_copy(xt_hbm.at[idx_vs[k4]], row_vs[k4], sem)
          for k4 in range(4)
      ]
      for cp in copies:
        cp.wait()

      @pl.loop(0, P)
      def _pix(pi):
        pidx = jnp.full((L,), pi, jnp.int32)
        ws = [plsc.load_gather(w_vs[k4], [pidx]) for k4 in range(4)]
        for cb in range(CB):
          sl = pl.ds(cb * L, L)
          acc = ws[0] * row_vs[0][pi, sl]
          acc = acc + ws[1] * row_vs[1][pi, sl]
          acc = acc + ws[2] * row_vs[2][pi, sl]
          acc = acc + ws[3] * row_vs[3][pi, sl]
          out_v[pi, sl] = acc

      pltpu.sync_copy(out_v, out_hbm.at[pl.ds(p0, P)])

  return k(xt, idx, w)


def kernel(x, warp):
  xf = x.astype(jnp.float32)
  gy = lax.broadcasted_iota(jnp.float32, (1, H, W), 1)
  gx = lax.broadcasted_iota(jnp.float32, (1, H, W), 2)
  sx = gx + warp[:, 0]
  sy = gy + warp[:, 1]
  x0 = jnp.floor(sx)
  y0 = jnp.floor(sy)
  wx = sx - x0
  wy = sy - y0

  def inb(xi, yi):
    return ((xi >= 0) & (xi <= W - 1) & (yi >= 0) & (yi <= H - 1)).astype(
        jnp.float32)

  boff = lax.broadcasted_iota(jnp.int32, (B, H, W), 0) * HW

  def flat_idx(xi, yi):
    xi_c = jnp.clip(xi, 0, W - 1).astype(jnp.int32)
    yi_c = jnp.clip(yi, 0, H - 1).astype(jnp.int32)
    return yi_c * W + xi_c + boff

  idx = jnp.stack([
      flat_idx(x0, y0), flat_idx(x0 + 1.0, y0),
      flat_idx(x0, y0 + 1.0), flat_idx(x0 + 1.0, y0 + 1.0)
  ]).reshape(4 * NPIX)
  w = jnp.stack([
      (1.0 - wx) * (1.0 - wy) * inb(x0, y0),
      wx * (1.0 - wy) * inb(x0 + 1.0, y0),
      (1.0 - wx) * wy * inb(x0, y0 + 1.0),
      wx * wy * inb(x0 + 1.0, y0 + 1.0),
  ]).reshape(4 * NPIX)

  xt = xf.transpose(0, 2, 3, 1).reshape(NPIX, C)
  xt = jnp.concatenate([xt, jnp.zeros((8, C), jnp.float32)], axis=0)
  out_t = _sc_gather_combine(xt, idx, w)
  return out_t.reshape(B, H, W, C).transpose(0, 3, 1, 2)


# speedup vs baseline: 8.1992x; 1.8097x over previous
import dataclasses
import functools

import jax
import jax.numpy as jnp
from jax import lax
from jax.experimental import pallas as pl
from jax.experimental.pallas import tpu as pltpu
from jax.experimental.pallas import tpu_sc as plsc

B, C, H, W = 2, 96, 384, 384
HW = H * W
NPIX = B * HW
NC, NS, L = 2, 16, 16
ROWS_W = H // NS
P = 128
CPR = W // P
NCHUNK = ROWS_W * CPR
G = P // L
CB = C // L


def _floor(v):
  t = v.astype(jnp.int32)
  tf = t.astype(jnp.float32)
  adj = jnp.where(tf > v, 1, 0)
  return t - adj, tf - adj.astype(jnp.float32)


def _sc_resample(xt, warp):
  mesh = plsc.VectorSubcoreMesh(core_axis_name="c", subcore_axis_name="s")
  cp = pltpu.CompilerParams()
  if "needs_layout_passes" in pltpu.CompilerParams.__dataclass_fields__:
    cp = dataclasses.replace(cp, needs_layout_passes=False)
  if "use_tc_tiling_on_sc" in pltpu.CompilerParams.__dataclass_fields__:
    cp = dataclasses.replace(cp, use_tc_tiling_on_sc=False)

  @functools.partial(
      pl.kernel,
      compiler_params=cp,
      out_type=jax.ShapeDtypeStruct((NPIX, C), jnp.float32),
      mesh=mesh,
      scratch_types=[
          [[pltpu.VMEM((P,), jnp.int32) for _ in range(4)] for _ in range(2)],
          [[pltpu.VMEM((P,), jnp.float32) for _ in range(4)] for _ in range(2)],
          [[pltpu.VMEM((P, C), jnp.float32) for _ in range(4)]
           for _ in range(2)],
          [pltpu.VMEM((P,), jnp.float32) for _ in range(2)],
          pltpu.VMEM((P, C), jnp.float32),
          [pltpu.SemaphoreType.DMA for _ in range(2)],
      ],
  )
  def k(xt_hbm, warp_hbm, out_hbm, idx_vs, w_vs, row_vs, wp_vs, out_v, sems):
    b = lax.axis_index("c")
    s = lax.axis_index("s")
    bb = b * HW
    woff0 = 2 * bb
    woff1 = 2 * bb + HW

    def stage(row, col0, st):
      q = row * W + col0
      pltpu.sync_copy(warp_hbm.at[pl.ds(woff0 + q, P)], wp_vs[0])
      pltpu.sync_copy(warp_hbm.at[pl.ds(woff1 + q, P)], wp_vs[1])
      rowf = row.astype(jnp.float32)
      for g in range(G):
        colf = (col0 + g * L).astype(jnp.float32)
        ii = lax.iota(jnp.int32, L).astype(jnp.float32)
        sl = pl.ds(g * L, L)
        sx = colf + ii + wp_vs[0][sl]
        sy = rowf + wp_vs[1][sl]
        x0i, x0f = _floor(sx)
        y0i, y0f = _floor(sy)
        wx = sx - x0f
        wy = sy - y0f
        bx0 = (x0f >= 0.0) & (x0f <= W - 1.0)
        bx1 = (x0f >= -1.0) & (x0f <= W - 2.0)
        by0 = (y0f >= 0.0) & (y0f <= H - 1.0)
        by1 = (y0f >= -1.0) & (y0f <= H - 2.0)
        ix0 = jnp.clip(x0i, 0, W - 1)
        ix1 = jnp.clip(x0i + 1, 0, W - 1)
        ry0 = bb + jnp.clip(y0i, 0, H - 1) * W
        ry1 = bb + jnp.clip(y0i + 1, 0, H - 1) * W
        idx_vs[st][0][sl] = ry0 + ix0
        idx_vs[st][1][sl] = ry0 + ix1
        idx_vs[st][2][sl] = ry1 + ix0
        idx_vs[st][3][sl] = ry1 + ix1
        zero = jnp.zeros((L,), jnp.float32)
        w_vs[st][0][sl] = jnp.where(bx0 & by0, (1.0 - wx) * (1.0 - wy), zero)
        w_vs[st][1][sl] = jnp.where(bx1 & by0, wx * (1.0 - wy), zero)
        w_vs[st][2][sl] = jnp.where(bx0 & by1, (1.0 - wx) * wy, zero)
        w_vs[st][3][sl] = jnp.where(bx1 & by1, wx * wy, zero)
      for k4 in range(4):
        pltpu.async_copy(xt_hbm.at[idx_vs[st][k4]], row_vs[st][k4], sems[st])

    def drain(st):
      for k4 in range(4):
        pltpu.make_async_copy(xt_hbm.at[idx_vs[st][k4]], row_vs[st][k4],
                              sems[st]).wait()

    def combine_out(row, col0, st):
      @plsc.parallel_loop(0, P, 1, unroll=2)
      def _pix(pi):
        pidx = jnp.full((L,), pi, jnp.int32)
        ws = [plsc.load_gather(w_vs[st][k4], [pidx]) for k4 in range(4)]
        for cb in range(CB):
          sl = pl.ds(cb * L, L)
          acc = ws[0] * row_vs[st][0][pi, sl]
          acc = acc + ws[1] * row_vs[st][1][pi, sl]
          acc = acc + ws[2] * row_vs[st][2][pi, sl]
          acc = acc + ws[3] * row_vs[st][3][pi, sl]
          out_v[pi, sl] = acc

      pltpu.sync_copy(out_v, out_hbm.at[pl.ds(bb + row * W + col0, P)])

    def rc(t):
      r3 = t // CPR
      return s * ROWS_W + r3, (t - r3 * CPR) * P

    r0, c0 = rc(jnp.int32(0))
    stage(r0, c0, 0)

    @pl.loop(0, NCHUNK, step=2)
    def _chunks(t):
      r1, c1 = rc(t + 1)
      stage(r1, c1, 1)
      ra, ca = rc(t)
      drain(0)
      combine_out(ra, ca, 0)

      @pl.when(t + 2 < NCHUNK)
      def _():
        r2, c2 = rc(t + 2)
        stage(r2, c2, 0)

      drain(1)
      combine_out(r1, c1, 1)

  return k(xt, warp)


def kernel(x, warp):
  xt = x.astype(jnp.float32).transpose(0, 2, 3, 1).reshape(NPIX, C)
  xt = jnp.concatenate([xt, jnp.zeros((8, C), jnp.float32)], axis=0)
  out_t = _sc_resample(xt, warp.astype(jnp.float32).reshape(2 * NPIX))
  return out_t.reshape(B, H, W, C).transpose(0, 3, 1, 2)
